# baseline (speedup 1.0000x reference)
import jax
import jax.numpy as jnp
from jax.experimental import pallas as pl
from jax.experimental.pallas import tpu as pltpu


def _dma_kernel(idx_ref, in_hbm, out_hbm, sem):
    C = out_hbm.shape[1]
    for c in range(C):
        src = idx_ref[c]
        pltpu.make_async_copy(
            in_hbm.at[:, src], out_hbm.at[:, c], sem
        ).start()
    for c in range(C):
        pltpu.make_async_copy(
            in_hbm.at[:, 0], out_hbm.at[:, 0], sem
        ).wait()


def kernel(input, indices):
    B, C, H, W = input.shape
    grid_spec = pltpu.PrefetchScalarGridSpec(
        num_scalar_prefetch=1,
        grid=(1,),
        in_specs=[pl.BlockSpec(memory_space=pltpu.MemorySpace.HBM)],
        out_specs=pl.BlockSpec(memory_space=pltpu.MemorySpace.HBM),
        scratch_shapes=[pltpu.SemaphoreType.DMA],
    )
    return pl.pallas_call(
        _dma_kernel,
        grid_spec=grid_spec,
        out_shape=jax.ShapeDtypeStruct(input.shape, input.dtype),
    )(indices, input)


# speedup vs baseline: 48.7826x; 48.7826x over previous
import jax
import jax.numpy as jnp
from jax.experimental import pallas as pl
from jax.experimental.pallas import tpu as pltpu

GC = 6


def _copy_kernel(idx_ref, *refs):
    in_refs, out_ref = refs[:-1], refs[-1]
    for k, in_ref in enumerate(in_refs):
        out_ref[:, k] = in_ref[:, 0]


def _make_in_spec(k, B, H, W):
    return pl.BlockSpec((B, 1, H, W), lambda i, idx: (0, idx[GC * i + k], 0, 0))


def kernel(input, indices):
    B, C, H, W = input.shape
    grid_spec = pltpu.PrefetchScalarGridSpec(
        num_scalar_prefetch=1,
        grid=(C // GC,),
        in_specs=[_make_in_spec(k, B, H, W) for k in range(GC)],
        out_specs=pl.BlockSpec((B, GC, H, W), lambda i, idx: (0, i, 0, 0)),
    )
    return pl.pallas_call(
        _copy_kernel,
        grid_spec=grid_spec,
        out_shape=jax.ShapeDtypeStruct(input.shape, input.dtype),
    )(indices, *([input] * GC))


# speedup vs baseline: 48.7873x; 1.0001x over previous
import jax
import jax.numpy as jnp
from jax.experimental import pallas as pl
from jax.experimental.pallas import tpu as pltpu

GC = 8


def _copy_kernel(idx_ref, *refs):
    in_refs, out_ref = refs[:-1], refs[-1]
    for k, in_ref in enumerate(in_refs):
        out_ref[:, k] = in_ref[:, 0]


def _make_in_spec(k, B, H, W):
    return pl.BlockSpec((B, 1, H, W), lambda i, idx: (0, idx[GC * i + k], 0, 0))


def kernel(input, indices):
    B, C, H, W = input.shape
    grid_spec = pltpu.PrefetchScalarGridSpec(
        num_scalar_prefetch=1,
        grid=(C // GC,),
        in_specs=[_make_in_spec(k, B, H, W) for k in range(GC)],
        out_specs=pl.BlockSpec((B, GC, H, W), lambda i, idx: (0, i, 0, 0)),
    )
    return pl.pallas_call(
        _copy_kernel,
        grid_spec=grid_spec,
        out_shape=jax.ShapeDtypeStruct(input.shape, input.dtype),
    )(indices, *([input] * GC))
